# baseline (speedup 1.0000x reference)
import functools

import jax
import jax.numpy as jnp
from jax import lax
from jax.experimental import pallas as pl
from jax.experimental.pallas import tpu as pltpu
from jax.experimental.pallas import tpu_sc as plsc

NUM_WORKERS = 32
CHUNK = 320


def _embedding_gather(idx2, table2):
    n = idx2.shape[0]
    half_v, two_d = table2.shape
    per_w = n // NUM_WORKERS
    k = per_w // CHUNK

    mesh = plsc.VectorSubcoreMesh(core_axis_name="c", subcore_axis_name="s")

    @functools.partial(
        pl.kernel,
        mesh=mesh,
        out_type=jax.ShapeDtypeStruct((n, two_d), jnp.float32),
        scratch_types=[
            pltpu.VMEM((per_w,), jnp.int32),
            pltpu.VMEM((CHUNK, two_d), jnp.float32),
            pltpu.VMEM((CHUNK, two_d), jnp.float32),
            pltpu.SemaphoreType.DMA,
            pltpu.SemaphoreType.DMA,
        ],
    )
    def body(idx_hbm, table_hbm, out_hbm, idx_v, buf0, buf1, gsem, osem):
        wid = lax.axis_index("s") * 2 + lax.axis_index("c")
        base = wid * per_w
        pltpu.sync_copy(idx_hbm.at[pl.ds(base, per_w)], idx_v)
        bufs = (buf0, buf1)
        cp0 = pltpu.async_copy(
            table_hbm.at[idx_v.at[pl.ds(0, CHUNK)]], bufs[0], gsem
        )
        pending = [cp0]
        out_pending = [None, None]
        for j in range(k):
            buf = bufs[j % 2]
            pending[0].wait()
            if j + 1 < k:
                nbuf = bufs[(j + 1) % 2]
                if out_pending[(j + 1) % 2] is not None:
                    out_pending[(j + 1) % 2].wait()
                    out_pending[(j + 1) % 2] = None
                pending[0] = pltpu.async_copy(
                    table_hbm.at[idx_v.at[pl.ds((j + 1) * CHUNK, CHUNK)]],
                    nbuf,
                    gsem,
                )
            out_pending[j % 2] = pltpu.async_copy(
                buf, out_hbm.at[pl.ds(base + j * CHUNK, CHUNK)], osem
            )
        for cp in out_pending:
            if cp is not None:
                cp.wait()

    return body(idx2, table2)


def kernel(x, table):
    b, l = x.shape
    v, d = table.shape
    idx = x.reshape(b * l).astype(jnp.int32)
    table2 = table.reshape(v // 2, d * 2)
    wide = _embedding_gather(lax.shift_right_logical(idx, 1), table2)
    odd = (idx & 1)[:, None].astype(jnp.bool_)
    out = jnp.where(odd, wide[:, d:], wide[:, :d])
    return out.reshape(b, l, d)


# speedup vs baseline: 1.0000x; 1.0000x over previous
import functools

import jax
import jax.numpy as jnp
from jax import lax
from jax.experimental import pallas as pl
from jax.experimental.pallas import tpu as pltpu
from jax.experimental.pallas import tpu_sc as plsc

NUM_WORKERS = 32
LANES = 16


def _embedding_gather(xT, table):
    l, b = xT.shape
    _, d = table.shape
    bw = b // NUM_WORKERS

    mesh = plsc.VectorSubcoreMesh(core_axis_name="c", subcore_axis_name="s")

    @functools.partial(
        pl.kernel,
        mesh=mesh,
        out_type=jax.ShapeDtypeStruct((l, d, b), jnp.float32),
        scratch_types=[
            pltpu.VMEM((l, bw), jnp.int32),
            pltpu.VMEM((bw, d), jnp.float32),
            pltpu.VMEM((bw, d), jnp.float32),
            pltpu.VMEM((d, bw), jnp.float32),
            pltpu.VMEM((d, bw), jnp.float32),
            pltpu.SemaphoreType.DMA,
            pltpu.SemaphoreType.DMA,
            pltpu.SemaphoreType.DMA,
            pltpu.SemaphoreType.DMA,
        ],
        compiler_params=pltpu.CompilerParams(
            use_tc_tiling_on_sc=False,
            needs_layout_passes=False,
        ),
    )
    def body(xT_hbm, table_hbm, out_hbm, xidx, wide0, wide1, slab0, slab1,
             gs0, gs1, os0, os1):
        wid = lax.axis_index("s") * 2 + lax.axis_index("c")
        b0 = wid * bw
        pltpu.sync_copy(xT_hbm.at[:, pl.ds(b0, bw)], xidx)

        wides = (wide0, wide1)
        slabs = (slab0, slab1)
        gsems = (gs0, gs1)
        osems = (os0, os1)

        bvecs = [k * LANES + lax.iota(jnp.int32, LANES)
                 for k in range(bw // LANES)]

        def start_gather(j):
            return pltpu.async_copy(
                table_hbm.at[xidx.at[j]], wides[j % 2], gsems[j % 2]
            )

        def transpose(j):
            wide = wides[j % 2]
            slab = slabs[j % 2]

            def col(c, _):
                cvec = jnp.full((LANES,), c, jnp.int32)
                row = slab.at[c]
                for k in range(bw // LANES):
                    row[pl.ds(k * LANES, LANES)] = plsc.load_gather(
                        wide, [bvecs[k], cvec]
                    )
                return _

            lax.fori_loop(0, d, col, None)

        gp = [start_gather(0), None]
        if l > 1:
            gp[1] = start_gather(1)
        op = [None, None]
        for j in range(l):
            s = j % 2
            gp[s].wait()
            if op[s] is not None:
                op[s].wait()
            transpose(j)
            if j + 2 < l:
                gp[s] = start_gather(j + 2)
            op[s] = pltpu.async_copy(
                slabs[s], out_hbm.at[j, :, pl.ds(b0, bw)], osems[s]
            )
        for cp in op:
            if cp is not None:
                cp.wait()

    return body(xT, table)


def kernel(x, table):
    b, l = x.shape
    _, d = table.shape
    xT = x.T.astype(jnp.int32)
    out3 = _embedding_gather(xT, table)
    return jnp.transpose(out3, (2, 0, 1))


# speedup vs baseline: 1.2497x; 1.2497x over previous
import functools

import jax
import jax.numpy as jnp
from jax import lax
from jax.experimental import pallas as pl
from jax.experimental.pallas import tpu as pltpu
from jax.experimental.pallas import tpu_sc as plsc

NUM_WORKERS = 32
CHUNK = 800


def _embedding_gather(idx, table):
    n = idx.shape[0]
    _, d = table.shape
    per_w = n // NUM_WORKERS
    k = per_w // CHUNK

    mesh = plsc.VectorSubcoreMesh(core_axis_name="c", subcore_axis_name="s")

    @functools.partial(
        pl.kernel,
        mesh=mesh,
        out_type=jax.ShapeDtypeStruct((n, d), jnp.float32),
        scratch_types=[
            pltpu.VMEM((per_w,), jnp.int32),
            pltpu.VMEM((CHUNK, d), jnp.float32),
            pltpu.VMEM((CHUNK, d), jnp.float32),
            pltpu.SemaphoreType.DMA,
            pltpu.SemaphoreType.DMA,
        ],
        compiler_params=pltpu.CompilerParams(use_tc_tiling_on_sc=False),
    )
    def body(idx_hbm, table_hbm, out_hbm, idx_v, buf0, buf1, gsem, osem):
        wid = lax.axis_index("s") * 2 + lax.axis_index("c")
        base = wid * per_w
        pltpu.sync_copy(idx_hbm.at[pl.ds(base, per_w)], idx_v)
        bufs = (buf0, buf1)
        cp0 = pltpu.async_copy(
            table_hbm.at[idx_v.at[pl.ds(0, CHUNK)]], bufs[0], gsem
        )
        pending = [cp0]
        out_pending = [None, None]
        for j in range(k):
            buf = bufs[j % 2]
            pending[0].wait()
            if j + 1 < k:
                nbuf = bufs[(j + 1) % 2]
                if out_pending[(j + 1) % 2] is not None:
                    out_pending[(j + 1) % 2].wait()
                    out_pending[(j + 1) % 2] = None
                pending[0] = pltpu.async_copy(
                    table_hbm.at[idx_v.at[pl.ds((j + 1) * CHUNK, CHUNK)]],
                    nbuf,
                    gsem,
                )
            out_pending[j % 2] = pltpu.async_copy(
                buf, out_hbm.at[pl.ds(base + j * CHUNK, CHUNK)], osem
            )
        for cp in out_pending:
            if cp is not None:
                cp.wait()

    return body(idx, table)


def kernel(x, table):
    b, l = x.shape
    _, d = table.shape
    idx = x.reshape(b * l).astype(jnp.int32)
    out = _embedding_gather(idx, table)
    return out.reshape(b, l, d)


# speedup vs baseline: 1.2574x; 1.0062x over previous
import functools

import jax
import jax.numpy as jnp
from jax import lax
from jax.experimental import pallas as pl
from jax.experimental.pallas import tpu as pltpu
from jax.experimental.pallas import tpu_sc as plsc

NUM_WORKERS = 32
CHUNK = 400
NBUF = 4


def _embedding_gather(idx, table):
    n = idx.shape[0]
    _, d = table.shape
    per_w = n // NUM_WORKERS
    k = per_w // CHUNK

    mesh = plsc.VectorSubcoreMesh(core_axis_name="c", subcore_axis_name="s")

    @functools.partial(
        pl.kernel,
        mesh=mesh,
        out_type=jax.ShapeDtypeStruct((n, d), jnp.float32),
        scratch_types=[
            pltpu.VMEM((per_w,), jnp.int32),
        ]
        + [pltpu.VMEM((CHUNK, d), jnp.float32) for _ in range(NBUF)]
        + [pltpu.SemaphoreType.DMA for _ in range(2 * NBUF)],
        compiler_params=pltpu.CompilerParams(use_tc_tiling_on_sc=False),
    )
    def body(idx_hbm, table_hbm, out_hbm, idx_v, *bufs_and_sems):
        bufs = bufs_and_sems[:NBUF]
        gsems = bufs_and_sems[NBUF:2 * NBUF]
        osems = bufs_and_sems[2 * NBUF:]
        wid = lax.axis_index("s") * 2 + lax.axis_index("c")
        base = wid * per_w
        pltpu.sync_copy(idx_hbm.at[pl.ds(base, per_w)], idx_v)

        def start_gather(j):
            return pltpu.async_copy(
                table_hbm.at[idx_v.at[pl.ds(j * CHUNK, CHUNK)]],
                bufs[j % NBUF],
                gsems[j % NBUF],
            )

        gp = [None] * NBUF
        op = [None] * NBUF
        for j in range(min(NBUF, k)):
            gp[j] = start_gather(j)
        for j in range(k):
            s = j % NBUF
            gp[s].wait()
            op[s] = pltpu.async_copy(
                bufs[s],
                out_hbm.at[pl.ds(base + j * CHUNK, CHUNK)],
                osems[s],
            )
            if j + NBUF < k:
                op[s].wait()
                gp[s] = start_gather(j + NBUF)
        for j in range(max(0, k - NBUF), k):
            s = j % NBUF
            if j + NBUF >= k and op[s] is not None:
                op[s].wait()

    return body(idx, table)


def kernel(x, table):
    b, l = x.shape
    _, d = table.shape
    idx = x.reshape(b * l).astype(jnp.int32)
    out = _embedding_gather(idx, table)
    return out.reshape(b, l, d)
